# baseline (speedup 1.0000x reference)
import functools

import jax
import jax.numpy as jnp
from jax.experimental import pallas as pl
from jax.experimental.pallas import tpu as pltpu


def _fused_body(tb, cap, nh, n_e, pre,
                x_ref, x2_ref, wg_ref, w1_ref, w2_ref,
                b1_ref, b2_ref, oa_ref, ob_ref, w8_ref, sv_ref, tri_ref,
                carry_ref):
    i = pl.program_id(0)

    @pl.when(i == 0)
    def _():
        carry_ref[0] = 0
        rows = jax.lax.broadcasted_iota(jnp.int32, (tb, tb), 0)
        cols = jax.lax.broadcasted_iota(jnp.int32, (tb, tb), 1)
        tri_ref[...] = (cols <= rows).astype(jnp.bfloat16)

    @pl.when(i < pre)
    def _():
        w2b = w2_ref[0]
        w2s = jnp.sum(w2b, axis=1, keepdims=True)
        pv = jax.lax.dot_general(w1_ref[0], w2s, (((1,), (0,)), ((), ())),
                                 preferred_element_type=jnp.float32)
        ps = jax.lax.dot_general(b1_ref[0], w2s, (((1,), (0,)), ((), ())),
                                 preferred_element_type=jnp.float32)
        e_idx = i // nh
        h_idx = i - e_idx * nh
        for e in range(n_e):
            c = n_e + e

            @pl.when(e_idx == e)
            def _():
                @pl.when(h_idx == 0)
                def _():
                    w8_ref[:, c:c + 1] = pv
                    sv_ref[0:1, e:e + 1] = (
                        ps + jnp.sum(b2_ref[0], axis=1, keepdims=True))

                @pl.when(h_idx != 0)
                def _():
                    w8_ref[:, c:c + 1] += pv
                    sv_ref[0:1, e:e + 1] += ps

        @pl.when(i == 0)
        def _():
            w8_ref[:, 0:n_e] = wg_ref[...]

    @pl.when(i >= pre)
    def _():
        j = i - pre

        def _half(x_half_ref, base_cnt, tok0):
            proj = jax.lax.dot_general(x_half_ref[...], w8_ref[...],
                                       (((1,), (0,)), ((), ())),
                                       preferred_element_type=jnp.float32)
            l0 = proj[:, 0:1]
            l1 = proj[:, 1:2]
            is1 = l1 > l0
            gate = jax.nn.sigmoid(jnp.abs(l1 - l0))
            ind1 = is1.astype(jnp.float32)
            c1 = jax.lax.dot_general(tri_ref[...], ind1.astype(jnp.bfloat16),
                                     (((1,), (0,)), ((), ())),
                                     preferred_element_type=jnp.float32)
            cnt1 = c1 + base_cnt
            gcnt = (jax.lax.broadcasted_iota(jnp.int32, (tb, 1), 0)
                    .astype(jnp.float32) + jnp.float32(1.0)
                    + tok0.astype(jnp.float32))
            pos = jnp.where(is1, cnt1 - 1.0, gcnt - cnt1 - 1.0)
            keep = (pos < jnp.float32(cap)).astype(jnp.float32)
            dsel = (jnp.where(is1, proj[:, 3:4], proj[:, 2:3])
                    + jnp.where(is1, sv_ref[0:1, 1:2], sv_ref[0:1, 0:1]))
            return gate * keep * dsel, jnp.sum(ind1)

        base = carry_ref[0].astype(jnp.float32)
        val_a, sum_a = _half(x_ref, base, 2 * j * tb)
        val_b, sum_b = _half(x2_ref, base + sum_a, (2 * j + 1) * tb)
        oa_ref[...] = val_a
        ob_ref[...] = val_b
        carry_ref[0] += (sum_a + sum_b).astype(jnp.int32)


def _lsm_body(z_ref, o_ref):
    z = z_ref[...]
    m = jnp.max(z, axis=1, keepdims=True)
    lse = m + jnp.log(jnp.sum(jnp.exp(z - m), axis=1, keepdims=True))
    o_ref[...] = z - lse


def kernel(input, wg, w1, b1, w2, b2):
    B, S, D = input.shape
    E = wg.shape[1]
    H = w1.shape[2]
    T = B * S
    cap = (T + E - 1) // E
    f32 = jnp.float32

    HB = 512
    TB = 512
    NH = H // HB
    PRE = E * NH
    NB = T // (2 * TB)
    xf = input.reshape(T, D)

    za, zb = pl.pallas_call(
        functools.partial(_fused_body, TB, cap, NH, E, PRE),
        grid=(PRE + NB,),
        in_specs=[
            pl.BlockSpec((TB, D), lambda i: (2 * jnp.maximum(i - PRE, 0), 0)),
            pl.BlockSpec((TB, D), lambda i: (2 * jnp.maximum(i - PRE, 0) + 1, 0)),
            pl.BlockSpec((D, E), lambda i: (0, 0)),
            pl.BlockSpec((1, D, HB),
                         lambda i: (jnp.where(i < PRE, i // NH, E - 1), 0,
                                    jnp.where(i < PRE, i % NH, NH - 1))),
            pl.BlockSpec((1, HB, D),
                         lambda i: (jnp.where(i < PRE, i // NH, E - 1),
                                    jnp.where(i < PRE, i % NH, NH - 1), 0)),
            pl.BlockSpec((1, 1, HB),
                         lambda i: (jnp.where(i < PRE, i // NH, E - 1), 0,
                                    jnp.where(i < PRE, i % NH, NH - 1))),
            pl.BlockSpec((1, 1, D),
                         lambda i: (jnp.where(i < PRE, i // NH, E - 1), 0, 0)),
        ],
        out_specs=[
            pl.BlockSpec((TB, 1), lambda i: (jnp.maximum(i - PRE, 0), 0)),
            pl.BlockSpec((TB, 1), lambda i: (jnp.maximum(i - PRE, 0), 0)),
        ],
        out_shape=[
            jax.ShapeDtypeStruct((T // 2, 1), f32),
            jax.ShapeDtypeStruct((T // 2, 1), f32),
        ],
        scratch_shapes=[
            pltpu.VMEM((D, 2 * E), f32),
            pltpu.VMEM((8, 128), f32),
            pltpu.VMEM((TB, TB), jnp.bfloat16),
            pltpu.SMEM((1,), jnp.int32),
        ],
    )(xf, xf, wg, w1, w2, b1.reshape(E, 1, H), b2.reshape(E, 1, D))

    z2 = jnp.stack([za.reshape(NB, TB), zb.reshape(NB, TB)],
                   axis=1).reshape(B, S)
    out = pl.pallas_call(
        _lsm_body,
        in_specs=[pl.BlockSpec((B, S), lambda: (0, 0))],
        out_specs=pl.BlockSpec((B, S), lambda: (0, 0)),
        out_shape=jax.ShapeDtypeStruct((B, S), f32),
    )(z2)
    return out
